# Initial kernel scaffold; baseline (speedup 1.0000x reference)
import jax
import jax.numpy as jnp
from jax.experimental import pallas as pl


def kernel(user_idx, item_idx, user_table, item_table, W):
    raise NotImplementedError("write your pallas kernel here")


# speedup vs baseline: 11.1646x; 11.1646x over previous
import functools

import jax
import jax.numpy as jnp
from jax import lax
from jax.experimental import pallas as pl
from jax.experimental.pallas import tpu as pltpu
from jax.experimental.pallas import tpu_sc as plsc

B = 4096
F_FIELDS = 26
E = 64
HIST = 20
G = 0.8
EMB = F_FIELDS * E
NB = B * F_FIELDS

NC = 2
NS = 16
NW = NC * NS
BAGS_W = NB // NW

C = 64
CHUNKS = BAGS_W // C
UC = 832
UCHUNKS = BAGS_W // UC


def _sc_gather(uidx, iidx, user_table, item_table):
    mesh = plsc.VectorSubcoreMesh(core_axis_name="c", subcore_axis_name="s")

    @functools.partial(
        pl.kernel,
        mesh=mesh,
        compiler_params=pltpu.CompilerParams(use_tc_tiling_on_sc=False),
        out_type=(
            jax.ShapeDtypeStruct((NB, E), jnp.float32),
            jax.ShapeDtypeStruct((NB, E), jnp.float32),
        ),
        scratch_types=[
            pltpu.VMEM((C * HIST,), jnp.int32),
            pltpu.VMEM((C * HIST, E), jnp.float32),
            pltpu.VMEM((C, E), jnp.float32),
            pltpu.SemaphoreType.DMA,
        ],
    )
    def k(uidx_hbm, iidx_hbm, utab_hbm, itab_hbm, a_out, b_out,
          idx_v, rows_v, acc_v, sem):
        wid = lax.axis_index("s") * NC + lax.axis_index("c")

        for uc in range(UCHUNKS):
            base = wid * BAGS_W + uc * UC
            pltpu.sync_copy(uidx_hbm.at[pl.ds(base, UC)],
                            idx_v.at[pl.ds(0, UC)])
            pltpu.async_copy(utab_hbm.at[idx_v.at[pl.ds(0, UC)]],
                             rows_v.at[pl.ds(0, UC)], sem).wait()
            pltpu.sync_copy(rows_v.at[pl.ds(0, UC)],
                            a_out.at[pl.ds(base, UC)])

        def chunk_body(t, carry):
            bag_base = wid * BAGS_W + t * C
            pltpu.sync_copy(iidx_hbm.at[pl.ds(bag_base * HIST, C * HIST)],
                            idx_v)
            pltpu.async_copy(itab_hbm.at[idx_v], rows_v, sem).wait()

            def bag_body(c, carry2):
                r0 = c * HIST
                for j in range(E // 16):
                    acc = rows_v[r0, pl.ds(j * 16, 16)]
                    for h in range(1, HIST):
                        acc = acc + rows_v[r0 + h, pl.ds(j * 16, 16)]
                    acc_v[c, pl.ds(j * 16, 16)] = acc
                return carry2

            lax.fori_loop(0, C, bag_body, 0)
            pltpu.sync_copy(acc_v, b_out.at[pl.ds(bag_base, C)])
            return carry

        lax.fori_loop(0, CHUNKS, chunk_body, 0)

    return k(uidx, iidx, user_table, item_table)


BM = 512


def _tc_combine(a, bsum, W):
    def body(a_ref, b_ref, w_ref, o_ref):
        y = lax.dot_general(b_ref[...], w_ref[...], (((1,), (1,)), ((), ())),
                            preferred_element_type=jnp.float32)
        z = G * a_ref[...] + ((1.0 - G) / HIST) * y
        ss = jnp.sum(z * z, axis=1, keepdims=True)
        o_ref[...] = z / jnp.maximum(jnp.sqrt(ss), 1e-12)

    return pl.pallas_call(
        body,
        grid=(B // BM,),
        in_specs=[
            pl.BlockSpec((BM, EMB), lambda i: (i, 0)),
            pl.BlockSpec((BM, EMB), lambda i: (i, 0)),
            pl.BlockSpec((EMB, EMB), lambda i: (0, 0)),
        ],
        out_specs=pl.BlockSpec((BM, EMB), lambda i: (i, 0)),
        out_shape=jax.ShapeDtypeStruct((B, EMB), jnp.float32),
    )(a, bsum, W)


def kernel(user_idx, item_idx, user_table, item_table, W):
    uidx = user_idx.reshape(-1).astype(jnp.int32)
    iidx = item_idx.reshape(-1).astype(jnp.int32)
    a_flat, bsum_flat = _sc_gather(uidx, iidx, user_table, item_table)
    a = a_flat.reshape(B, EMB)
    bsum = bsum_flat.reshape(B, EMB)
    return _tc_combine(a, bsum, W)


# speedup vs baseline: 15.7315x; 1.4091x over previous
import functools

import jax
import jax.numpy as jnp
from jax import lax
from jax.experimental import pallas as pl
from jax.experimental.pallas import tpu as pltpu
from jax.experimental.pallas import tpu_sc as plsc

B = 4096
F_FIELDS = 26
E = 64
HIST = 20
G = 0.8
EMB = F_FIELDS * E
NB = B * F_FIELDS

NC = 2
NS = 16
NW = NC * NS
BAGS_W = NB // NW

C = 32
CH = C * HIST
CHUNKS = BAGS_W // C
UC = 416
UCHUNKS = BAGS_W // UC


def _tree_sum(vals):
    while len(vals) > 1:
        nxt = [a + b for a, b in zip(vals[0::2], vals[1::2])]
        if len(vals) % 2:
            nxt.append(vals[-1])
        vals = nxt
    return vals[0]


def _sc_gather(uidx, iidx, user_table, item_table):
    mesh = plsc.VectorSubcoreMesh(core_axis_name="c", subcore_axis_name="s")

    @functools.partial(
        pl.kernel,
        mesh=mesh,
        compiler_params=pltpu.CompilerParams(use_tc_tiling_on_sc=False),
        out_type=(
            jax.ShapeDtypeStruct((NB, E), jnp.float32),
            jax.ShapeDtypeStruct((NB, E), jnp.float32),
        ),
        scratch_types=[
            pltpu.VMEM((CH,), jnp.int32),
            pltpu.VMEM((CH,), jnp.int32),
            pltpu.VMEM((CH, E), jnp.float32),
            pltpu.VMEM((CH, E), jnp.float32),
            pltpu.VMEM((C, E), jnp.float32),
            pltpu.SemaphoreType.DMA,
            pltpu.SemaphoreType.DMA,
        ],
    )
    def k(uidx_hbm, iidx_hbm, utab_hbm, itab_hbm, a_out, b_out,
          idx0, idx1, rows0, rows1, acc_v, sem0, sem1):
        wid = lax.axis_index("s") * NC + lax.axis_index("c")
        bag_base = wid * BAGS_W

        idx_bufs = (idx0, idx1)
        row_bufs = (rows0, rows1)
        sems = (sem0, sem1)

        def issue_item(chunk, p):
            pltpu.sync_copy(iidx_hbm.at[pl.ds((bag_base + chunk * C) * HIST, CH)],
                            idx_bufs[p])
            pltpu.make_async_copy(itab_hbm.at[idx_bufs[p]], row_bufs[p],
                                  sems[p]).start()

        def drain_item(p):
            pltpu.make_async_copy(itab_hbm.at[idx_bufs[p]], row_bufs[p],
                                  sems[p]).wait()

        def reduce_chunk(chunk, p):
            rows_v = row_bufs[p]

            def bag_body(c, carry):
                r0 = c * HIST
                for j in range(E // 16):
                    s = pl.ds(j * 16, 16)
                    acc_v[c, s] = _tree_sum(
                        [rows_v[r0 + h, s] for h in range(HIST)])
                return carry

            lax.fori_loop(0, C, bag_body, 0)
            pltpu.sync_copy(acc_v, b_out.at[pl.ds(bag_base + chunk * C, C)])

        issue_item(0, 0)
        T = CHUNKS // 2

        def iter_body(t, carry):
            c0 = 2 * t
            issue_item(c0 + 1, 1)
            drain_item(0)
            reduce_chunk(c0, 0)

            @pl.when(t < T - 1)
            def _():
                issue_item(c0 + 2, 0)

            drain_item(1)
            reduce_chunk(c0 + 1, 1)
            return carry

        lax.fori_loop(0, T, iter_body, 0)

        def issue_user(chunk, p):
            base = bag_base + chunk * UC
            pltpu.sync_copy(uidx_hbm.at[pl.ds(base, UC)],
                            idx_bufs[p].at[pl.ds(0, UC)])
            pltpu.make_async_copy(utab_hbm.at[idx_bufs[p].at[pl.ds(0, UC)]],
                                  row_bufs[p].at[pl.ds(0, UC)], sems[p]).start()

        issue_user(0, 0)
        issue_user(1, 1)
        for uc in range(UCHUNKS):
            p = uc % 2
            pltpu.make_async_copy(utab_hbm.at[idx_bufs[p].at[pl.ds(0, UC)]],
                                  row_bufs[p].at[pl.ds(0, UC)], sems[p]).wait()
            pltpu.sync_copy(row_bufs[p].at[pl.ds(0, UC)],
                            a_out.at[pl.ds(bag_base + uc * UC, UC)])
            if uc + 2 < UCHUNKS:
                issue_user(uc + 2, p)

    return k(uidx, iidx, user_table, item_table)


BM = 512


def _tc_combine(a, bsum, W):
    def body(a_ref, b_ref, w_ref, o_ref):
        y = lax.dot_general(b_ref[...], w_ref[...], (((1,), (1,)), ((), ())),
                            preferred_element_type=jnp.float32)
        z = G * a_ref[...] + ((1.0 - G) / HIST) * y
        ss = jnp.sum(z * z, axis=1, keepdims=True)
        o_ref[...] = z / jnp.maximum(jnp.sqrt(ss), 1e-12)

    return pl.pallas_call(
        body,
        grid=(B // BM,),
        in_specs=[
            pl.BlockSpec((BM, EMB), lambda i: (i, 0)),
            pl.BlockSpec((BM, EMB), lambda i: (i, 0)),
            pl.BlockSpec((EMB, EMB), lambda i: (0, 0)),
        ],
        out_specs=pl.BlockSpec((BM, EMB), lambda i: (i, 0)),
        out_shape=jax.ShapeDtypeStruct((B, EMB), jnp.float32),
    )(a, bsum, W)


def kernel(user_idx, item_idx, user_table, item_table, W):
    uidx = user_idx.reshape(-1).astype(jnp.int32)
    iidx = item_idx.reshape(-1).astype(jnp.int32)
    a_flat, bsum_flat = _sc_gather(uidx, iidx, user_table, item_table)
    a = a_flat.reshape(B, EMB)
    bsum = bsum_flat.reshape(B, EMB)
    return _tc_combine(a, bsum, W)


# speedup vs baseline: 17.2667x; 1.0976x over previous
import functools

import jax
import jax.numpy as jnp
from jax import lax
from jax.experimental import pallas as pl
from jax.experimental.pallas import tpu as pltpu
from jax.experimental.pallas import tpu_sc as plsc

B = 4096
F_FIELDS = 26
E = 64
HIST = 20
G = 0.8
EMB = F_FIELDS * E
NB = B * F_FIELDS

NC = 2
NS = 16
NW = NC * NS
BAGS_W = NB // NW

C = 64
CH = C * HIST
CHUNKS = BAGS_W // C
UC = 832
UCHUNKS = BAGS_W // UC

LB = 32


def _tree_sum(vals):
    while len(vals) > 1:
        nxt = [a + b for a, b in zip(vals[0::2], vals[1::2])]
        if len(vals) % 2:
            nxt.append(vals[-1])
        vals = nxt
    return vals[0]


def _sc_gather(uidx, iidx, user_table, item_table):
    mesh = plsc.VectorSubcoreMesh(core_axis_name="c", subcore_axis_name="s")

    @functools.partial(
        pl.kernel,
        mesh=mesh,
        compiler_params=pltpu.CompilerParams(use_tc_tiling_on_sc=False),
        out_type=(
            jax.ShapeDtypeStruct((NB, E), jnp.bfloat16),
            jax.ShapeDtypeStruct((NB, E), jnp.bfloat16),
        ),
        scratch_types=[
            pltpu.VMEM((CH,), jnp.int32),
            pltpu.VMEM((CH,), jnp.int32),
            pltpu.VMEM((CH, E), jnp.bfloat16),
            pltpu.VMEM((CH, E), jnp.bfloat16),
            pltpu.VMEM((C, E), jnp.bfloat16),
            pltpu.SemaphoreType.DMA,
            pltpu.SemaphoreType.DMA,
        ],
    )
    def k(uidx_hbm, iidx_hbm, utab_hbm, itab_hbm, a_out, b_out,
          idx0, idx1, rows0, rows1, acc_v, sem0, sem1):
        wid = lax.axis_index("s") * NC + lax.axis_index("c")
        bag_base = wid * BAGS_W

        idx_bufs = (idx0, idx1)
        row_bufs = (rows0, rows1)
        sems = (sem0, sem1)

        def issue_item(chunk, p):
            pltpu.sync_copy(iidx_hbm.at[pl.ds((bag_base + chunk * C) * HIST, CH)],
                            idx_bufs[p])
            pltpu.make_async_copy(itab_hbm.at[idx_bufs[p]], row_bufs[p],
                                  sems[p]).start()

        def drain_item(p):
            pltpu.make_async_copy(itab_hbm.at[idx_bufs[p]], row_bufs[p],
                                  sems[p]).wait()

        def reduce_chunk(chunk, p):
            rows_v = row_bufs[p]

            def bag_body(c, carry):
                r0 = c * HIST
                for j in range(E // LB):
                    s = pl.ds(j * LB, LB)
                    acc_v[c, s] = _tree_sum(
                        [rows_v[r0 + h, s] for h in range(HIST)])
                return carry

            lax.fori_loop(0, C, bag_body, 0)
            pltpu.sync_copy(acc_v, b_out.at[pl.ds(bag_base + chunk * C, C)])

        issue_item(0, 0)
        T = CHUNKS // 2

        def iter_body(t, carry):
            c0 = 2 * t
            issue_item(c0 + 1, 1)
            drain_item(0)
            reduce_chunk(c0, 0)

            @pl.when(t < T - 1)
            def _():
                issue_item(c0 + 2, 0)

            drain_item(1)
            reduce_chunk(c0 + 1, 1)
            return carry

        lax.fori_loop(0, T, iter_body, 0)

        def issue_user(chunk, p):
            base = bag_base + chunk * UC
            pltpu.sync_copy(uidx_hbm.at[pl.ds(base, UC)],
                            idx_bufs[p].at[pl.ds(0, UC)])
            pltpu.make_async_copy(utab_hbm.at[idx_bufs[p].at[pl.ds(0, UC)]],
                                  row_bufs[p].at[pl.ds(0, UC)], sems[p]).start()

        issue_user(0, 0)
        issue_user(1, 1)
        for uc in range(UCHUNKS):
            p = uc % 2
            pltpu.make_async_copy(utab_hbm.at[idx_bufs[p].at[pl.ds(0, UC)]],
                                  row_bufs[p].at[pl.ds(0, UC)], sems[p]).wait()
            pltpu.sync_copy(row_bufs[p].at[pl.ds(0, UC)],
                            a_out.at[pl.ds(bag_base + uc * UC, UC)])
            if uc + 2 < UCHUNKS:
                issue_user(uc + 2, p)

    return k(uidx, iidx, user_table, item_table)


BM = 512


def _tc_combine(a, bsum, W):
    def body(a_ref, b_ref, w_ref, o_ref):
        y = lax.dot_general(b_ref[...], w_ref[...], (((1,), (1,)), ((), ())),
                            preferred_element_type=jnp.float32)
        z = G * a_ref[...].astype(jnp.float32) + ((1.0 - G) / HIST) * y
        ss = jnp.sum(z * z, axis=1, keepdims=True)
        o_ref[...] = z / jnp.maximum(jnp.sqrt(ss), 1e-12)

    return pl.pallas_call(
        body,
        grid=(B // BM,),
        in_specs=[
            pl.BlockSpec((BM, EMB), lambda i: (i, 0)),
            pl.BlockSpec((BM, EMB), lambda i: (i, 0)),
            pl.BlockSpec((EMB, EMB), lambda i: (0, 0)),
        ],
        out_specs=pl.BlockSpec((BM, EMB), lambda i: (i, 0)),
        out_shape=jax.ShapeDtypeStruct((B, EMB), jnp.float32),
    )(a, bsum, W)


def kernel(user_idx, item_idx, user_table, item_table, W):
    uidx = user_idx.reshape(-1).astype(jnp.int32)
    iidx = item_idx.reshape(-1).astype(jnp.int32)
    utab = user_table.astype(jnp.bfloat16)
    itab = item_table.astype(jnp.bfloat16)
    a_flat, bsum_flat = _sc_gather(uidx, iidx, utab, itab)
    a = a_flat.reshape(B, EMB)
    bsum = bsum_flat.reshape(B, EMB)
    return _tc_combine(a, bsum, W.astype(jnp.bfloat16))
